# Initial kernel scaffold; baseline (speedup 1.0000x reference)
import jax
import jax.numpy as jnp
from jax.experimental import pallas as pl


def kernel(inputs, edge_index, params):
    raise NotImplementedError("write your pallas kernel here")


# speedup vs baseline: 382.0706x; 382.0706x over previous
import functools

import jax
import jax.numpy as jnp
from jax import lax
from jax.experimental import pallas as pl
from jax.experimental.pallas import tpu as pltpu
from jax.experimental.pallas import tpu_sc as plsc

N = 100000
E = 3200000
NC, NS, L = 2, 16, 16
NW = NC * NS
SUB = 128
BLK = 512
NSUB = BLK // SUB
NBLK = 196
EPT = NBLK * BLK
E_PAD = EPT * NW
N_ACC = 100352
RPT = N_ACC // NS
NROW = 1024
NGRID = 98

@functools.cache
def _mesh():
    return plsc.VectorSubcoreMesh(core_axis_name="c", subcore_axis_name="s",
                                  num_cores=NC, num_subcores=NS)


def _leaky02(x):
    return jnp.where(x > 0, x, 0.2 * x)


def _sc_pass1_body(tsrc_hbm, tdst_hbm, src_hbm, dst_hbm, ab_hbm, z_hbm,
                   out_hbm, sidx, didx, gsrc, gdst, contrib, abv, acc,
                   gsem, ssem):
    c = lax.axis_index("c")
    s = lax.axis_index("s")
    wid = c * NS + s

    pltpu.sync_copy(z_hbm, acc.at[pl.ds(s * RPT, RPT)])
    pltpu.sync_copy(ab_hbm, abv)
    plsc.subcore_barrier()

    def group(g, carry2):
        rowv = lax.iota(jnp.int32, L) + g * L
        cols = [jnp.full((L,), f, jnp.int32) for f in range(16)]
        g1 = [plsc.load_gather(gsrc, [rowv, cols[f]]) for f in range(12)]
        g2 = [plsc.load_gather(gdst, [rowv, cols[f]]) for f in range(12)]
        e = [_leaky02(g1[f] + g2[f]) for f in range(12)]
        la0 = abv[0] * e[0]
        la1 = abv[1] * e[1]
        ld0 = abv[2] * e[2]
        for f in range(3, 7):
            ld0 = ld0 + abv[f] * e[f]
        ld1 = abv[7] * e[7]
        for f in range(8, 12):
            ld1 = ld1 + abv[f] * e[f]
        wa0 = jnp.exp(la0)
        wa1 = jnp.exp(la1)
        wd0 = jnp.exp(ld0)
        wd1 = jnp.exp(ld1)
        out = [wa0, wa1, wa0 * g1[0], wa1 * g1[1], wd0, wd1]
        out += [wd0 * g1[2 + f] for f in range(5)]
        out += [wd1 * g1[7 + f] for f in range(5)]
        for ci in range(16):
            plsc.store_scatter(contrib, [rowv, cols[ci]], out[ci])
        return carry2

    def block(b, carry):
        row0 = wid * (EPT // SUB) + b * NSUB
        pltpu.sync_copy(src_hbm.at[pl.ds(row0, NSUB)], sidx)
        pltpu.sync_copy(dst_hbm.at[pl.ds(row0, NSUB)], didx)
        gd = [pltpu.async_copy(
                  tsrc_hbm.at[sidx.at[j]], gsrc.at[pl.ds(j * SUB, SUB)], gsem)
              for j in range(NSUB)]
        gd += [pltpu.async_copy(
                   tdst_hbm.at[didx.at[j]], gdst.at[pl.ds(j * SUB, SUB)], gsem)
               for j in range(NSUB)]
        adescs = []
        for q in range(NSUB):
            gd[q].wait()
            gd[NSUB + q].wait()
            lax.fori_loop(q * (SUB // L), (q + 1) * (SUB // L), group, 0)
            adescs.append(pltpu.async_copy(
                contrib.at[pl.ds(q * SUB, SUB)], acc.at[didx.at[q]], ssem,
                add=True))
        for d in adescs:
            d.wait()
        return carry

    lax.fori_loop(0, NBLK, block, 0)
    plsc.subcore_barrier()
    pltpu.sync_copy(acc.at[pl.ds(s * RPT, RPT)],
                    out_hbm.at[c].at[pl.ds(s * RPT, RPT)])


@functools.cache
def _sc_pass1_kernel():
    return pl.kernel(
        _sc_pass1_body,
        out_type=jax.ShapeDtypeStruct((NC, N_ACC, 16), jnp.float32),
        mesh=_mesh(),
        scratch_types=[
        pltpu.VMEM((NSUB, SUB), jnp.int32),
        pltpu.VMEM((NSUB, SUB), jnp.int32),
        pltpu.VMEM((BLK, 16), jnp.float32),
        pltpu.VMEM((BLK, 16), jnp.float32),
        pltpu.VMEM((BLK, 16), jnp.float32),
        pltpu.VMEM((12, 16), jnp.float32),
            pltpu.VMEM_SHARED((N_ACC, 16), jnp.float32),
            pltpu.SemaphoreType.DMA,
            pltpu.SemaphoreType.DMA,
        ],
        compiler_params=pltpu.CompilerParams(
            needs_layout_passes=False, use_tc_tiling_on_sc=False),
    )


def _sc_pass1(*args):
    return _sc_pass1_kernel()(*args)


def _sc_pass2_body(t2_hbm, src_hbm, dst_hbm, ab_hbm, z_hbm,
                   out_hbm, sidx, didx, gsrc, gdst, contrib, abv, acc,
                   gsem, ssem):
    c = lax.axis_index("c")
    s = lax.axis_index("s")
    wid = c * NS + s

    pltpu.sync_copy(z_hbm, acc.at[pl.ds(s * RPT, RPT)])
    pltpu.sync_copy(ab_hbm, abv)
    plsc.subcore_barrier()

    def group(g, carry2):
        rowv = lax.iota(jnp.int32, L) + g * L
        cols = [jnp.full((L,), f, jnp.int32) for f in range(8)]
        g1 = [plsc.load_gather(gsrc, [rowv, cols[f]]) for f in range(4)]
        g2 = [plsc.load_gather(gdst, [rowv, cols[4 + f]]) for f in range(4)]
        e = [_leaky02(g1[f] + g2[f]) for f in range(4)]
        l0 = abv[0] * e[0] + abv[1] * e[1]
        l1 = abv[2] * e[2] + abv[3] * e[3]
        w0 = jnp.exp(l0)
        w1 = jnp.exp(l1)
        zero = jnp.zeros((L,), jnp.float32)
        out = [w0, w1, w0 * g1[0], w0 * g1[1], w1 * g1[2], w1 * g1[3],
               zero, zero]
        for ci in range(8):
            plsc.store_scatter(contrib, [rowv, cols[ci]], out[ci])
        return carry2

    def block(b, carry):
        row0 = wid * (EPT // SUB) + b * NSUB
        pltpu.sync_copy(src_hbm.at[pl.ds(row0, NSUB)], sidx)
        pltpu.sync_copy(dst_hbm.at[pl.ds(row0, NSUB)], didx)
        gd = [pltpu.async_copy(
                  t2_hbm.at[sidx.at[j]], gsrc.at[pl.ds(j * SUB, SUB)], gsem)
              for j in range(NSUB)]
        gd += [pltpu.async_copy(
                   t2_hbm.at[didx.at[j]], gdst.at[pl.ds(j * SUB, SUB)], gsem)
               for j in range(NSUB)]
        adescs = []
        for q in range(NSUB):
            gd[q].wait()
            gd[NSUB + q].wait()
            lax.fori_loop(q * (SUB // L), (q + 1) * (SUB // L), group, 0)
            adescs.append(pltpu.async_copy(
                contrib.at[pl.ds(q * SUB, SUB)], acc.at[didx.at[q]], ssem,
                add=True))
        for d in adescs:
            d.wait()
        return carry

    lax.fori_loop(0, NBLK, block, 0)
    plsc.subcore_barrier()
    pltpu.sync_copy(acc.at[pl.ds(s * RPT, RPT)],
                    out_hbm.at[c].at[pl.ds(s * RPT, RPT)])


@functools.cache
def _sc_pass2_kernel():
    return pl.kernel(
        _sc_pass2_body,
        out_type=jax.ShapeDtypeStruct((NC, N_ACC, 8), jnp.float32),
        mesh=_mesh(),
        scratch_types=[
        pltpu.VMEM((NSUB, SUB), jnp.int32),
        pltpu.VMEM((NSUB, SUB), jnp.int32),
        pltpu.VMEM((BLK, 8), jnp.float32),
        pltpu.VMEM((BLK, 8), jnp.float32),
        pltpu.VMEM((BLK, 8), jnp.float32),
        pltpu.VMEM((4, 16), jnp.float32),
            pltpu.VMEM_SHARED((N_ACC, 8), jnp.float32),
            pltpu.SemaphoreType.DMA,
            pltpu.SemaphoreType.DMA,
        ],
        compiler_params=pltpu.CompilerParams(
            needs_layout_passes=False, use_tc_tiling_on_sc=False),
    )


def _sc_pass2(*args):
    return _sc_pass2_kernel()(*args)


def _t1_body(x_ref, ws_ref, bs_ref, wd_ref, bd_ref, wr_ref, br_ref,
             tsrc_ref, tdst_ref, res_ref):
    x5 = x_ref[:, :5]
    tsrc_ref[...] = jnp.dot(x5, ws_ref[...],
                            preferred_element_type=jnp.float32) + bs_ref[...]
    tdst_ref[...] = jnp.dot(x5, wd_ref[...],
                            preferred_element_type=jnp.float32) + bd_ref[...]
    res_ref[...] = jnp.dot(x5, wr_ref[...],
                           preferred_element_type=jnp.float32) + br_ref[...]


def _elu(x):
    return jnp.where(x > 0, x, jnp.exp(x) - 1.0)


def _t2_body(p_ref, res_ref, wc_ref, bc_ref, hatt_ref, t2_ref, res2_ref):
    pa = p_ref[0] + p_ref[1]
    den_a = pa[:, 0:2] + 1e-9
    rst_a = pa[:, 2:4]
    den_d = pa[:, 4:6] + 1e-9
    rst_d = pa[:, 6:16]
    h_att = _elu(rst_a / den_a + res_ref[:, 0:2])
    den_dx = jnp.concatenate(
        [jnp.broadcast_to(den_d[:, 0:1], (NROW, 5)),
         jnp.broadcast_to(den_d[:, 1:2], (NROW, 5))], axis=1)
    h_def = _elu(rst_d / den_dx + res_ref[:, 2:12])
    hw = jnp.dot(h_def, wc_ref[...],
                 preferred_element_type=jnp.float32) + bc_ref[...]
    hatt_ref[...] = h_att
    t2_ref[...] = hw[:, 0:8]
    res2_ref[...] = hw[:, 8:12]


def _t3_body(p_ref, res2_ref, hatt_ref, x_ref,
             w1_ref, b1_ref, w2_ref, b2_ref, w3_ref, b3_ref, w4_ref, b4_ref,
             out_ref):
    pa = p_ref[0] + p_ref[1]
    den = pa[:, 0:2] + 1e-9
    rst = pa[:, 2:6]
    den_x = jnp.concatenate(
        [jnp.broadcast_to(den[:, 0:1], (NROW, 2)),
         jnp.broadcast_to(den[:, 1:2], (NROW, 2))], axis=1)
    h_def2 = _elu(rst / den_x + res2_ref[...])
    z = jnp.concatenate(
        [hatt_ref[...], h_def2, x_ref[...],
         jnp.zeros((NROW, 2), jnp.float32)], axis=1)

    def lk(v):
        return jnp.where(v > 0, v, 0.01 * v)

    h = lk(jnp.dot(z, w1_ref[...],
                   preferred_element_type=jnp.float32) + b1_ref[...])
    h = lk(jnp.dot(h, w2_ref[...],
                   preferred_element_type=jnp.float32) + b2_ref[...])
    h = lk(jnp.dot(h, w3_ref[...],
                   preferred_element_type=jnp.float32) + b3_ref[...])
    o = jnp.dot(h, w4_ref[...],
                preferred_element_type=jnp.float32) + b4_ref[...]
    out_ref[...] = 1.0 / (1.0 + jnp.exp(-o[:, 0:1]))


def _row_spec(w):
    return pl.BlockSpec((NROW, w), lambda i: (i, 0))


def _full_spec(shape):
    return pl.BlockSpec(shape, lambda i: tuple(0 for _ in shape))


def kernel(inputs, edge_index, params):
    p = params
    f32 = jnp.float32

    src = edge_index[0].astype(jnp.int32)
    dst = edge_index[1].astype(jnp.int32)
    pad = E_PAD - E
    src_r = jnp.concatenate([src, jnp.zeros((pad,), jnp.int32)]
                            ).reshape(E_PAD // SUB, SUB)
    dst_r = jnp.concatenate([dst, jnp.full((pad,), N, jnp.int32)]
                            ).reshape(E_PAD // SUB, SUB)

    ws1 = jnp.concatenate([p['Ws_a'], p['Ws_d1'],
                           jnp.zeros((5, 4), f32)], axis=1)
    bs1 = jnp.concatenate([p['bs_a'], p['bs_d1'],
                           jnp.zeros((4,), f32)]).reshape(1, 16)
    wd1 = jnp.concatenate([p['Wd_a'], p['Wd_d1'],
                           jnp.zeros((5, 4), f32)], axis=1)
    bd1 = jnp.concatenate([p['bd_a'], p['bd_d1'],
                           jnp.zeros((4,), f32)]).reshape(1, 16)
    wr1 = jnp.concatenate([p['Wr_a'], p['Wr_d1']], axis=1)
    br1 = jnp.concatenate([p['br_a'], p['br_d1']]).reshape(1, 12)

    ab1 = jnp.concatenate([p['attn_a'].reshape(-1),
                           p['attn_d1'].reshape(-1)])
    ab1 = jnp.broadcast_to(ab1[:, None], (12, 16)).astype(f32)
    ab2 = jnp.broadcast_to(p['attn_d2'].reshape(-1)[:, None], (4, 16))

    wc = jnp.concatenate([p['Ws_d2'], p['Wd_d2'], p['Wr_d2']], axis=1)
    bc = jnp.concatenate([p['bs_d2'], p['bd_d2'], p['br_d2']]).reshape(1, 12)

    w1p = jnp.zeros((16, 256), f32).at[:14, :196].set(p['W1'])
    b1p = jnp.zeros((1, 256), f32).at[0, :196].set(p['b1'])
    w2p = jnp.zeros((256, 256), f32).at[:196, :196].set(p['W2'])
    b2p = jnp.zeros((1, 256), f32).at[0, :196].set(p['b2'])
    w3p = jnp.zeros((256, 16), f32).at[:196, :14].set(p['W3'])
    b3p = jnp.zeros((1, 16), f32).at[0, :14].set(p['b3'])
    w4p = jnp.zeros((16, 8), f32).at[:14, 0:1].set(p['W4'])
    b4p = jnp.zeros((1, 8), f32).at[0, 0:1].set(p['b4'])

    z16 = jnp.zeros((RPT, 16), f32)
    z8 = jnp.zeros((RPT, 8), f32)

    tsrc, tdst, res1 = pl.pallas_call(
        _t1_body,
        grid=(NGRID,),
        in_specs=[_row_spec(8), _full_spec((5, 16)), _full_spec((1, 16)),
                  _full_spec((5, 16)), _full_spec((1, 16)),
                  _full_spec((5, 12)), _full_spec((1, 12))],
        out_specs=[_row_spec(16), _row_spec(16), _row_spec(12)],
        out_shape=[jax.ShapeDtypeStruct((N_ACC, 16), f32),
                   jax.ShapeDtypeStruct((N_ACC, 16), f32),
                   jax.ShapeDtypeStruct((N, 12), f32)],
    )(inputs, ws1, bs1, wd1, bd1, wr1, br1)

    p1 = _sc_pass1(tsrc, tdst, src_r, dst_r, ab1, z16)

    hatt, t2, res2 = pl.pallas_call(
        _t2_body,
        grid=(NGRID,),
        in_specs=[pl.BlockSpec((NC, NROW, 16), lambda i: (0, i, 0)),
                  _row_spec(12), _full_spec((10, 12)), _full_spec((1, 12))],
        out_specs=[_row_spec(2), _row_spec(8), _row_spec(4)],
        out_shape=[jax.ShapeDtypeStruct((N, 2), f32),
                   jax.ShapeDtypeStruct((N_ACC, 8), f32),
                   jax.ShapeDtypeStruct((N, 4), f32)],
    )(p1, res1, wc, bc)

    p2 = _sc_pass2(t2, src_r, dst_r, ab2, z8)

    out = pl.pallas_call(
        _t3_body,
        grid=(NGRID,),
        in_specs=[pl.BlockSpec((NC, NROW, 8), lambda i: (0, i, 0)),
                  _row_spec(4), _row_spec(2), _row_spec(8),
                  _full_spec((16, 256)), _full_spec((1, 256)),
                  _full_spec((256, 256)), _full_spec((1, 256)),
                  _full_spec((256, 16)), _full_spec((1, 16)),
                  _full_spec((16, 8)), _full_spec((1, 8))],
        out_specs=[_row_spec(1)],
        out_shape=[jax.ShapeDtypeStruct((N, 1), f32)],
    )(p2, res2, hatt, inputs, w1p, b1p, w2p, b2p, w3p, b3p, w4p, b4p)

    return out[0]


# speedup vs baseline: 501.3441x; 1.3122x over previous
import functools

import jax
import jax.numpy as jnp
from jax import lax
from jax.experimental import pallas as pl
from jax.experimental.pallas import tpu as pltpu
from jax.experimental.pallas import tpu_sc as plsc

N = 100000
E = 3200000
NC, NS, L = 2, 16, 16
NW = NC * NS
SUB = 128
BLK = 256
NSUB = BLK // SUB
EPT = 100352
NBLK = EPT // BLK
E_PAD = EPT * NW
N_ACC = 100352
RPT = N_ACC // NS
NROW = 1024
NGRID = 98


@functools.cache
def _mesh():
    return plsc.VectorSubcoreMesh(core_axis_name="c", subcore_axis_name="s",
                                  num_cores=NC, num_subcores=NS)


def _leaky02(x):
    return jnp.where(x > 0, x, 0.2 * x)


def _compute_group1(gs, gd, dpv, contrib, abv, g):
    rowv = lax.iota(jnp.int32, L) + g * L
    cols = [jnp.full((L,), f, jnp.int32) for f in range(16)]
    g1 = [plsc.load_gather(gs, [dpv, rowv, cols[f]]) for f in range(12)]
    g2 = [plsc.load_gather(gd, [dpv, rowv, cols[f]]) for f in range(12)]
    e = [_leaky02(g1[f] + g2[f]) for f in range(12)]
    la0 = abv[0] * e[0]
    la1 = abv[1] * e[1]
    ld0 = abv[2] * e[2]
    for f in range(3, 7):
        ld0 = ld0 + abv[f] * e[f]
    ld1 = abv[7] * e[7]
    for f in range(8, 12):
        ld1 = ld1 + abv[f] * e[f]
    wa0 = jnp.exp(la0)
    wa1 = jnp.exp(la1)
    wd0 = jnp.exp(ld0)
    wd1 = jnp.exp(ld1)
    out = [wa0, wa1, wa0 * g1[0], wa1 * g1[1], wd0, wd1]
    out += [wd0 * g1[2 + f] for f in range(5)]
    out += [wd1 * g1[7 + f] for f in range(5)]
    for ci in range(16):
        plsc.store_scatter(contrib, [rowv, cols[ci]], out[ci])


def _compute_group2(gs, gd, dpv, contrib, abv, g):
    rowv = lax.iota(jnp.int32, L) + g * L
    cols = [jnp.full((L,), f, jnp.int32) for f in range(8)]
    g1 = [plsc.load_gather(gs, [dpv, rowv, cols[f]]) for f in range(4)]
    g2 = [plsc.load_gather(gd, [dpv, rowv, cols[4 + f]]) for f in range(4)]
    e = [_leaky02(g1[f] + g2[f]) for f in range(4)]
    l0 = abv[0] * e[0] + abv[1] * e[1]
    l1 = abv[2] * e[2] + abv[3] * e[3]
    w0 = jnp.exp(l0)
    w1 = jnp.exp(l1)
    zero = jnp.zeros((L,), jnp.float32)
    out = [w0, w1, w0 * g1[0], w0 * g1[1], w1 * g1[2], w1 * g1[3],
           zero, zero]
    for ci in range(8):
        plsc.store_scatter(contrib, [rowv, cols[ci]], out[ci])


def _edge_body(W, compute_group):
    def body(tsrc_hbm, tdst_hbm, idx_hbm, ab_hbm, z_hbm, out_hbm,
             ib, gs, gd, contrib, abv, acc, isem, gsem, ssem):
        c = lax.axis_index("c")
        s = lax.axis_index("s")
        wid = c * NS + s

        pltpu.sync_copy(z_hbm, acc.at[pl.ds(s * RPT, RPT)])
        pltpu.sync_copy(ab_hbm, abv)
        plsc.subcore_barrier()
        base_row = wid * (EPT // SUB)
        pltpu.sync_copy(idx_hbm.at[pl.ds(base_row, NSUB)], ib.at[0])

        def drain_idx():
            pltpu.make_async_copy(
                idx_hbm.at[pl.ds(0, NSUB)], ib.at[0], isem).wait()

        def drain_gathers():
            pltpu.make_async_copy(
                tsrc_hbm.at[pl.ds(0, BLK)], gs.at[0], gsem).wait()
            pltpu.make_async_copy(
                tsrc_hbm.at[pl.ds(0, BLK)], gd.at[0], gsem).wait()

        def drain_scatters():
            pltpu.make_async_copy(
                tsrc_hbm.at[pl.ds(0, BLK)], contrib, ssem).wait()

        def step(i, carry):
            dp = lax.rem(i, 2)
            sl_cur = lax.rem(i, 3)
            sl_nxt = lax.rem(i + 1, 3)
            sl_prv = lax.rem(i + 2, 3)

            @pl.when(i >= 2)
            def _():
                drain_scatters()

            @pl.when(i >= 1)
            def _():
                drain_gathers()

            @pl.when(i <= NBLK - 1)
            def _():
                @pl.when(i >= 1)
                def _():
                    drain_idx()

                rown = base_row + jnp.minimum(i + 1, NBLK - 1) * NSUB
                pltpu.async_copy(idx_hbm.at[pl.ds(rown, NSUB)],
                                 ib.at[sl_nxt], isem)
                for q in range(NSUB):
                    pltpu.async_copy(
                        tsrc_hbm.at[ib.at[sl_cur, q, 0]],
                        gs.at[dp].at[pl.ds(q * SUB, SUB)], gsem)
                    pltpu.async_copy(
                        tdst_hbm.at[ib.at[sl_cur, q, 1]],
                        gd.at[dp].at[pl.ds(q * SUB, SUB)], gsem)

            @pl.when(i >= 1)
            def _():
                dpv = jnp.full((L,), 1 - dp, jnp.int32)

                def grp(g, cc):
                    compute_group(gs, gd, dpv, contrib, abv, g)
                    return cc

                for q in range(NSUB):
                    lax.fori_loop(q * (SUB // L), (q + 1) * (SUB // L),
                                  grp, 0)
                    pltpu.async_copy(contrib.at[pl.ds(q * SUB, SUB)],
                                     acc.at[ib.at[sl_prv, q, 1]], ssem,
                                     add=True)

            return carry

        lax.fori_loop(0, NBLK + 1, step, 0)
        drain_scatters()
        drain_idx()
        plsc.subcore_barrier()
        pltpu.sync_copy(acc.at[pl.ds(s * RPT, RPT)],
                        out_hbm.at[c].at[pl.ds(s * RPT, RPT)])

    return body


@functools.cache
def _sc_pass_kernel(W, nab, pass_id):
    compute_group = _compute_group1 if pass_id == 1 else _compute_group2
    return pl.kernel(
        _edge_body(W, compute_group),
        out_type=jax.ShapeDtypeStruct((NC, N_ACC, W), jnp.float32),
        mesh=_mesh(),
        scratch_types=[
            pltpu.VMEM((3, NSUB, 2, SUB), jnp.int32),
            pltpu.VMEM((2, BLK, W), jnp.float32),
            pltpu.VMEM((2, BLK, W), jnp.float32),
            pltpu.VMEM((BLK, W), jnp.float32),
            pltpu.VMEM((nab, 16), jnp.float32),
            pltpu.VMEM_SHARED((N_ACC, W), jnp.float32),
            pltpu.SemaphoreType.DMA,
            pltpu.SemaphoreType.DMA,
            pltpu.SemaphoreType.DMA,
        ],
        compiler_params=pltpu.CompilerParams(
            needs_layout_passes=False, use_tc_tiling_on_sc=False),
    )


def _sc_pass1(tsrc, tdst, idx_sd, ab, z):
    return _sc_pass_kernel(16, 12, 1)(tsrc, tdst, idx_sd, ab, z)


def _sc_pass2(t2, idx_sd, ab, z):
    return _sc_pass_kernel(8, 4, 2)(t2, t2, idx_sd, ab, z)


def _t1_body(x_ref, ws_ref, bs_ref, wd_ref, bd_ref, wr_ref, br_ref,
             tsrc_ref, tdst_ref, res_ref):
    x5 = x_ref[:, :5]
    tsrc_ref[...] = jnp.dot(x5, ws_ref[...],
                            preferred_element_type=jnp.float32) + bs_ref[...]
    tdst_ref[...] = jnp.dot(x5, wd_ref[...],
                            preferred_element_type=jnp.float32) + bd_ref[...]
    res_ref[...] = jnp.dot(x5, wr_ref[...],
                           preferred_element_type=jnp.float32) + br_ref[...]


def _elu(x):
    return jnp.where(x > 0, x, jnp.exp(x) - 1.0)


def _t2_body(p_ref, res_ref, wc_ref, bc_ref, hatt_ref, t2_ref, res2_ref):
    pa = p_ref[0] + p_ref[1]
    den_a = pa[:, 0:2] + 1e-9
    rst_a = pa[:, 2:4]
    den_d = pa[:, 4:6] + 1e-9
    rst_d = pa[:, 6:16]
    h_att = _elu(rst_a / den_a + res_ref[:, 0:2])
    den_dx = jnp.concatenate(
        [jnp.broadcast_to(den_d[:, 0:1], (NROW, 5)),
         jnp.broadcast_to(den_d[:, 1:2], (NROW, 5))], axis=1)
    h_def = _elu(rst_d / den_dx + res_ref[:, 2:12])
    hw = jnp.dot(h_def, wc_ref[...],
                 preferred_element_type=jnp.float32) + bc_ref[...]
    hatt_ref[...] = h_att
    t2_ref[...] = hw[:, 0:8]
    res2_ref[...] = hw[:, 8:12]


def _t3_body(p_ref, res2_ref, hatt_ref, x_ref,
             w1_ref, b1_ref, w2_ref, b2_ref, w3_ref, b3_ref, w4_ref, b4_ref,
             out_ref):
    pa = p_ref[0] + p_ref[1]
    den = pa[:, 0:2] + 1e-9
    rst = pa[:, 2:6]
    den_x = jnp.concatenate(
        [jnp.broadcast_to(den[:, 0:1], (NROW, 2)),
         jnp.broadcast_to(den[:, 1:2], (NROW, 2))], axis=1)
    h_def2 = _elu(rst / den_x + res2_ref[...])
    z = jnp.concatenate(
        [hatt_ref[...], h_def2, x_ref[...],
         jnp.zeros((NROW, 2), jnp.float32)], axis=1)

    def lk(v):
        return jnp.where(v > 0, v, 0.01 * v)

    h = lk(jnp.dot(z, w1_ref[...],
                   preferred_element_type=jnp.float32) + b1_ref[...])
    h = lk(jnp.dot(h, w2_ref[...],
                   preferred_element_type=jnp.float32) + b2_ref[...])
    h = lk(jnp.dot(h, w3_ref[...],
                   preferred_element_type=jnp.float32) + b3_ref[...])
    o = jnp.dot(h, w4_ref[...],
                preferred_element_type=jnp.float32) + b4_ref[...]
    out_ref[...] = 1.0 / (1.0 + jnp.exp(-o[:, 0:1]))


def _row_spec(w):
    return pl.BlockSpec((NROW, w), lambda i: (i, 0))


def _full_spec(shape):
    return pl.BlockSpec(shape, lambda i: tuple(0 for _ in shape))


def kernel(inputs, edge_index, params):
    p = params
    f32 = jnp.float32

    src = edge_index[0].astype(jnp.int32)
    dst = edge_index[1].astype(jnp.int32)
    pad = E_PAD - E
    src_r = jnp.concatenate([src, jnp.zeros((pad,), jnp.int32)]
                            ).reshape(E_PAD // SUB, SUB)
    dst_r = jnp.concatenate([dst, jnp.full((pad,), N, jnp.int32)]
                            ).reshape(E_PAD // SUB, SUB)
    idx_sd = jnp.stack([src_r, dst_r], axis=1)

    ws1 = jnp.concatenate([p['Ws_a'], p['Ws_d1'],
                           jnp.zeros((5, 4), f32)], axis=1)
    bs1 = jnp.concatenate([p['bs_a'], p['bs_d1'],
                           jnp.zeros((4,), f32)]).reshape(1, 16)
    wd1 = jnp.concatenate([p['Wd_a'], p['Wd_d1'],
                           jnp.zeros((5, 4), f32)], axis=1)
    bd1 = jnp.concatenate([p['bd_a'], p['bd_d1'],
                           jnp.zeros((4,), f32)]).reshape(1, 16)
    wr1 = jnp.concatenate([p['Wr_a'], p['Wr_d1']], axis=1)
    br1 = jnp.concatenate([p['br_a'], p['br_d1']]).reshape(1, 12)

    ab1 = jnp.concatenate([p['attn_a'].reshape(-1),
                           p['attn_d1'].reshape(-1)])
    ab1 = jnp.broadcast_to(ab1[:, None], (12, 16)).astype(f32)
    ab2 = jnp.broadcast_to(p['attn_d2'].reshape(-1)[:, None], (4, 16))

    wc = jnp.concatenate([p['Ws_d2'], p['Wd_d2'], p['Wr_d2']], axis=1)
    bc = jnp.concatenate([p['bs_d2'], p['bd_d2'], p['br_d2']]).reshape(1, 12)

    w1p = jnp.zeros((16, 256), f32).at[:14, :196].set(p['W1'])
    b1p = jnp.zeros((1, 256), f32).at[0, :196].set(p['b1'])
    w2p = jnp.zeros((256, 256), f32).at[:196, :196].set(p['W2'])
    b2p = jnp.zeros((1, 256), f32).at[0, :196].set(p['b2'])
    w3p = jnp.zeros((256, 16), f32).at[:196, :14].set(p['W3'])
    b3p = jnp.zeros((1, 16), f32).at[0, :14].set(p['b3'])
    w4p = jnp.zeros((16, 8), f32).at[:14, 0:1].set(p['W4'])
    b4p = jnp.zeros((1, 8), f32).at[0, 0:1].set(p['b4'])

    z16 = jnp.zeros((RPT, 16), f32)
    z8 = jnp.zeros((RPT, 8), f32)

    tsrc, tdst, res1 = pl.pallas_call(
        _t1_body,
        grid=(NGRID,),
        in_specs=[_row_spec(8), _full_spec((5, 16)), _full_spec((1, 16)),
                  _full_spec((5, 16)), _full_spec((1, 16)),
                  _full_spec((5, 12)), _full_spec((1, 12))],
        out_specs=[_row_spec(16), _row_spec(16), _row_spec(12)],
        out_shape=[jax.ShapeDtypeStruct((N_ACC, 16), f32),
                   jax.ShapeDtypeStruct((N_ACC, 16), f32),
                   jax.ShapeDtypeStruct((N, 12), f32)],
    )(inputs, ws1, bs1, wd1, bd1, wr1, br1)

    p1 = _sc_pass1(tsrc, tdst, idx_sd, ab1, z16)

    hatt, t2, res2 = pl.pallas_call(
        _t2_body,
        grid=(NGRID,),
        in_specs=[pl.BlockSpec((NC, NROW, 16), lambda i: (0, i, 0)),
                  _row_spec(12), _full_spec((10, 12)), _full_spec((1, 12))],
        out_specs=[_row_spec(2), _row_spec(8), _row_spec(4)],
        out_shape=[jax.ShapeDtypeStruct((N, 2), f32),
                   jax.ShapeDtypeStruct((N_ACC, 8), f32),
                   jax.ShapeDtypeStruct((N, 4), f32)],
    )(p1, res1, wc, bc)

    p2 = _sc_pass2(t2, idx_sd, ab2, z8)

    out = pl.pallas_call(
        _t3_body,
        grid=(NGRID,),
        in_specs=[pl.BlockSpec((NC, NROW, 8), lambda i: (0, i, 0)),
                  _row_spec(4), _row_spec(2), _row_spec(8),
                  _full_spec((16, 256)), _full_spec((1, 256)),
                  _full_spec((256, 256)), _full_spec((1, 256)),
                  _full_spec((256, 16)), _full_spec((1, 16)),
                  _full_spec((16, 8)), _full_spec((1, 8))],
        out_specs=[_row_spec(1)],
        out_shape=[jax.ShapeDtypeStruct((N, 1), f32)],
    )(p2, res2, hatt, inputs, w1p, b1p, w2p, b2p, w3p, b3p, w4p, b4p)

    return out[0]


# speedup vs baseline: 578.2599x; 1.1534x over previous
import functools

import jax
import jax.numpy as jnp
from jax import lax
from jax.experimental import pallas as pl
from jax.experimental.pallas import tpu as pltpu
from jax.experimental.pallas import tpu_sc as plsc

N = 100000
E = 3200000
NC, NS, L = 2, 16, 16
NW = NC * NS
SUB = 128
EPT = 100352
E_PAD = EPT * NW
N_ACC = 100352
RPT = N_ACC // NS
NROW = 2048
NGRID = 49


@functools.cache
def _mesh():
    return plsc.VectorSubcoreMesh(core_axis_name="c", subcore_axis_name="s",
                                  num_cores=NC, num_subcores=NS)


def _leaky02(x):
    return jnp.where(x > 0, x, 0.2 * x)


def _compute_group1(gs, gd, dpv, contrib, abv, g):
    rowv = lax.iota(jnp.int32, L) + g * L
    cols = [jnp.full((L,), f, jnp.int32) for f in range(16)]
    g1 = [plsc.load_gather(gs, [dpv, rowv, cols[f]]) for f in range(12)]
    g2 = [plsc.load_gather(gd, [dpv, rowv, cols[f]]) for f in range(12)]
    e = [_leaky02(g1[f] + g2[f]) for f in range(12)]
    la0 = abv[0] * e[0]
    la1 = abv[1] * e[1]
    ld0 = abv[2] * e[2]
    for f in range(3, 7):
        ld0 = ld0 + abv[f] * e[f]
    ld1 = abv[7] * e[7]
    for f in range(8, 12):
        ld1 = ld1 + abv[f] * e[f]
    wa0 = jnp.exp(la0)
    wa1 = jnp.exp(la1)
    wd0 = jnp.exp(ld0)
    wd1 = jnp.exp(ld1)
    out = [wa0, wa1, wa0 * g1[0], wa1 * g1[1], wd0, wd1]
    out += [wd0 * g1[2 + f] for f in range(5)]
    out += [wd1 * g1[7 + f] for f in range(5)]
    for ci in range(16):
        plsc.store_scatter(contrib, [rowv, cols[ci]], out[ci])


def _compute_group2(gs, gd, dpv, contrib, abv, g):
    rowv = lax.iota(jnp.int32, L) + g * L
    cols = [jnp.full((L,), f, jnp.int32) for f in range(8)]
    g1 = [plsc.load_gather(gs, [dpv, rowv, cols[f]]) for f in range(4)]
    g2 = [plsc.load_gather(gd, [dpv, rowv, cols[4 + f]]) for f in range(4)]
    e = [_leaky02(g1[f] + g2[f]) for f in range(4)]
    l0 = abv[0] * e[0] + abv[1] * e[1]
    l1 = abv[2] * e[2] + abv[3] * e[3]
    w0 = jnp.exp(l0)
    w1 = jnp.exp(l1)
    zero = jnp.zeros((L,), jnp.float32)
    out = [w0, w1, w0 * g1[0], w0 * g1[1], w1 * g1[2], w1 * g1[3],
           zero, zero]
    for ci in range(8):
        plsc.store_scatter(contrib, [rowv, cols[ci]], out[ci])


def _edge_body(W, compute_group, BLK, NSUB, NBLK):
    def body(tsrc_hbm, tdst_hbm, idx_hbm, ab_hbm, z_hbm, out_hbm,
             ib, gs, gd, contrib, abv, acc, isem, gsem, ssem):
        c = lax.axis_index("c")
        s = lax.axis_index("s")
        wid = c * NS + s

        pltpu.sync_copy(z_hbm, acc.at[pl.ds(s * RPT, RPT)])
        pltpu.sync_copy(ab_hbm, abv)
        plsc.subcore_barrier()
        base_row = wid * (EPT // SUB)
        pltpu.sync_copy(idx_hbm.at[pl.ds(base_row, NSUB)], ib.at[0])

        def drain_idx():
            pltpu.make_async_copy(
                idx_hbm.at[pl.ds(0, NSUB)], ib.at[0], isem).wait()

        def drain_gathers():
            pltpu.make_async_copy(
                tsrc_hbm.at[pl.ds(0, BLK)], gs.at[0], gsem).wait()
            pltpu.make_async_copy(
                tsrc_hbm.at[pl.ds(0, BLK)], gd.at[0], gsem).wait()

        def drain_scatters():
            pltpu.make_async_copy(
                tsrc_hbm.at[pl.ds(0, BLK)], contrib, ssem).wait()

        def step(i, carry):
            dp = lax.rem(i, 2)
            sl_cur = lax.rem(i, 3)
            sl_nxt = lax.rem(i + 1, 3)
            sl_prv = lax.rem(i + 2, 3)

            @pl.when(i >= 2)
            def _():
                drain_scatters()

            @pl.when(i >= 1)
            def _():
                drain_gathers()

            @pl.when(i <= NBLK - 1)
            def _():
                @pl.when(i >= 1)
                def _():
                    drain_idx()

                rown = base_row + jnp.minimum(i + 1, NBLK - 1) * NSUB
                pltpu.async_copy(idx_hbm.at[pl.ds(rown, NSUB)],
                                 ib.at[sl_nxt], isem)
                for q in range(NSUB):
                    pltpu.async_copy(
                        tsrc_hbm.at[ib.at[sl_cur, q, 0]],
                        gs.at[dp].at[pl.ds(q * SUB, SUB)], gsem)
                    pltpu.async_copy(
                        tdst_hbm.at[ib.at[sl_cur, q, 1]],
                        gd.at[dp].at[pl.ds(q * SUB, SUB)], gsem)

            @pl.when(i >= 1)
            def _():
                dpv = jnp.full((L,), 1 - dp, jnp.int32)

                def grp(g, cc):
                    compute_group(gs, gd, dpv, contrib, abv, g)
                    return cc

                for q in range(NSUB):
                    lax.fori_loop(q * (SUB // L), (q + 1) * (SUB // L),
                                  grp, 0)
                    pltpu.async_copy(contrib.at[pl.ds(q * SUB, SUB)],
                                     acc.at[ib.at[sl_prv, q, 1]], ssem,
                                     add=True)

            return carry

        lax.fori_loop(0, NBLK + 1, step, 0)
        drain_scatters()
        drain_idx()
        plsc.subcore_barrier()
        pltpu.sync_copy(acc.at[pl.ds(s * RPT, RPT)],
                        out_hbm.at[c].at[pl.ds(s * RPT, RPT)])

    return body


@functools.cache
def _sc_pass_kernel(W, nab, pass_id, blk):
    compute_group = _compute_group1 if pass_id == 1 else _compute_group2
    BLK, NSUB, NBLK = blk, blk // SUB, EPT // blk
    return pl.kernel(
        _edge_body(W, compute_group, BLK, NSUB, NBLK),
        out_type=jax.ShapeDtypeStruct((NC, N_ACC, W), jnp.float32),
        mesh=_mesh(),
        scratch_types=[
            pltpu.VMEM((3, NSUB, 2, SUB), jnp.int32),
            pltpu.VMEM((2, BLK, W), jnp.float32),
            pltpu.VMEM((2, BLK, W), jnp.float32),
            pltpu.VMEM((BLK, W), jnp.float32),
            pltpu.VMEM((nab, 16), jnp.float32),
            pltpu.VMEM_SHARED((N_ACC, W), jnp.float32),
            pltpu.SemaphoreType.DMA,
            pltpu.SemaphoreType.DMA,
            pltpu.SemaphoreType.DMA,
        ],
        compiler_params=pltpu.CompilerParams(
            needs_layout_passes=False, use_tc_tiling_on_sc=False),
    )


def _sc_pass1(tsrc, tdst, idx_sd, ab, z):
    return _sc_pass_kernel(16, 12, 1, 256)(tsrc, tdst, idx_sd, ab, z)


def _sc_pass2(t2, idx_sd, ab, z):
    return _sc_pass_kernel(8, 4, 2, 1024)(t2, t2, idx_sd, ab, z)


def _t1_body(x_ref, ws_ref, bs_ref, wd_ref, bd_ref, wr_ref, br_ref,
             tsrc_ref, tdst_ref, res_ref):
    x5 = x_ref[:, :5]
    tsrc_ref[...] = jnp.dot(x5, ws_ref[...],
                            preferred_element_type=jnp.float32) + bs_ref[...]
    tdst_ref[...] = jnp.dot(x5, wd_ref[...],
                            preferred_element_type=jnp.float32) + bd_ref[...]
    res_ref[...] = jnp.dot(x5, wr_ref[...],
                           preferred_element_type=jnp.float32) + br_ref[...]


def _elu(x):
    return jnp.where(x > 0, x, jnp.exp(x) - 1.0)


def _t2_body(p_ref, res_ref, wc_ref, bc_ref, hatt_ref, t2_ref, res2_ref):
    pa = p_ref[0] + p_ref[1]
    den_a = pa[:, 0:2] + 1e-9
    rst_a = pa[:, 2:4]
    den_d = pa[:, 4:6] + 1e-9
    rst_d = pa[:, 6:16]
    h_att = _elu(rst_a / den_a + res_ref[:, 0:2])
    den_dx = jnp.concatenate(
        [jnp.broadcast_to(den_d[:, 0:1], (NROW, 5)),
         jnp.broadcast_to(den_d[:, 1:2], (NROW, 5))], axis=1)
    h_def = _elu(rst_d / den_dx + res_ref[:, 2:12])
    hw = jnp.dot(h_def, wc_ref[...],
                 preferred_element_type=jnp.float32) + bc_ref[...]
    hatt_ref[...] = h_att
    t2_ref[...] = hw[:, 0:8]
    res2_ref[...] = hw[:, 8:12]


def _t3_body(p_ref, res2_ref, hatt_ref, x_ref,
             w1_ref, b1_ref, w2_ref, b2_ref, w3_ref, b3_ref, w4_ref, b4_ref,
             out_ref):
    pa = p_ref[0] + p_ref[1]
    den = pa[:, 0:2] + 1e-9
    rst = pa[:, 2:6]
    den_x = jnp.concatenate(
        [jnp.broadcast_to(den[:, 0:1], (NROW, 2)),
         jnp.broadcast_to(den[:, 1:2], (NROW, 2))], axis=1)
    h_def2 = _elu(rst / den_x + res2_ref[...])
    z = jnp.concatenate(
        [hatt_ref[...], h_def2, x_ref[...],
         jnp.zeros((NROW, 2), jnp.float32)], axis=1)

    def lk(v):
        return jnp.where(v > 0, v, 0.01 * v)

    h = lk(jnp.dot(z, w1_ref[...],
                   preferred_element_type=jnp.float32) + b1_ref[...])
    h = lk(jnp.dot(h, w2_ref[...],
                   preferred_element_type=jnp.float32) + b2_ref[...])
    h = lk(jnp.dot(h, w3_ref[...],
                   preferred_element_type=jnp.float32) + b3_ref[...])
    o = jnp.dot(h, w4_ref[...],
                preferred_element_type=jnp.float32) + b4_ref[...]
    out_ref[...] = 1.0 / (1.0 + jnp.exp(-o[:, 0:1]))


def _row_spec(w):
    return pl.BlockSpec((NROW, w), lambda i: (i, 0))


def _full_spec(shape):
    return pl.BlockSpec(shape, lambda i: tuple(0 for _ in shape))


def kernel(inputs, edge_index, params):
    p = params
    f32 = jnp.float32

    src = edge_index[0].astype(jnp.int32)
    dst = edge_index[1].astype(jnp.int32)
    pad = E_PAD - E
    src_r = jnp.concatenate([src, jnp.zeros((pad,), jnp.int32)]
                            ).reshape(E_PAD // SUB, SUB)
    dst_r = jnp.concatenate([dst, jnp.full((pad,), N, jnp.int32)]
                            ).reshape(E_PAD // SUB, SUB)
    idx_sd = jnp.stack([src_r, dst_r], axis=1)

    ws1 = jnp.concatenate([p['Ws_a'], p['Ws_d1'],
                           jnp.zeros((5, 4), f32)], axis=1)
    bs1 = jnp.concatenate([p['bs_a'], p['bs_d1'],
                           jnp.zeros((4,), f32)]).reshape(1, 16)
    wd1 = jnp.concatenate([p['Wd_a'], p['Wd_d1'],
                           jnp.zeros((5, 4), f32)], axis=1)
    bd1 = jnp.concatenate([p['bd_a'], p['bd_d1'],
                           jnp.zeros((4,), f32)]).reshape(1, 16)
    wr1 = jnp.concatenate([p['Wr_a'], p['Wr_d1']], axis=1)
    br1 = jnp.concatenate([p['br_a'], p['br_d1']]).reshape(1, 12)

    ab1 = jnp.concatenate([p['attn_a'].reshape(-1),
                           p['attn_d1'].reshape(-1)])
    ab1 = jnp.broadcast_to(ab1[:, None], (12, 16)).astype(f32)
    ab2 = jnp.broadcast_to(p['attn_d2'].reshape(-1)[:, None], (4, 16))

    wc = jnp.concatenate([p['Ws_d2'], p['Wd_d2'], p['Wr_d2']], axis=1)
    bc = jnp.concatenate([p['bs_d2'], p['bd_d2'], p['br_d2']]).reshape(1, 12)

    w1p = jnp.zeros((16, 256), f32).at[:14, :196].set(p['W1'])
    b1p = jnp.zeros((1, 256), f32).at[0, :196].set(p['b1'])
    w2p = jnp.zeros((256, 256), f32).at[:196, :196].set(p['W2'])
    b2p = jnp.zeros((1, 256), f32).at[0, :196].set(p['b2'])
    w3p = jnp.zeros((256, 16), f32).at[:196, :14].set(p['W3'])
    b3p = jnp.zeros((1, 16), f32).at[0, :14].set(p['b3'])
    w4p = jnp.zeros((16, 8), f32).at[:14, 0:1].set(p['W4'])
    b4p = jnp.zeros((1, 8), f32).at[0, 0:1].set(p['b4'])

    z16 = jnp.zeros((RPT, 16), f32)
    z8 = jnp.zeros((RPT, 8), f32)

    tsrc, tdst, res1 = pl.pallas_call(
        _t1_body,
        grid=(NGRID,),
        in_specs=[_row_spec(8), _full_spec((5, 16)), _full_spec((1, 16)),
                  _full_spec((5, 16)), _full_spec((1, 16)),
                  _full_spec((5, 12)), _full_spec((1, 12))],
        out_specs=[_row_spec(16), _row_spec(16), _row_spec(12)],
        out_shape=[jax.ShapeDtypeStruct((N_ACC, 16), f32),
                   jax.ShapeDtypeStruct((N_ACC, 16), f32),
                   jax.ShapeDtypeStruct((N, 12), f32)],
    )(inputs, ws1, bs1, wd1, bd1, wr1, br1)

    p1 = _sc_pass1(tsrc, tdst, idx_sd, ab1, z16)

    hatt, t2, res2 = pl.pallas_call(
        _t2_body,
        grid=(NGRID,),
        in_specs=[pl.BlockSpec((NC, NROW, 16), lambda i: (0, i, 0)),
                  _row_spec(12), _full_spec((10, 12)), _full_spec((1, 12))],
        out_specs=[_row_spec(2), _row_spec(8), _row_spec(4)],
        out_shape=[jax.ShapeDtypeStruct((N, 2), f32),
                   jax.ShapeDtypeStruct((N_ACC, 8), f32),
                   jax.ShapeDtypeStruct((N, 4), f32)],
    )(p1, res1, wc, bc)

    p2 = _sc_pass2(t2, idx_sd, ab2, z8)

    out = pl.pallas_call(
        _t3_body,
        grid=(NGRID,),
        in_specs=[pl.BlockSpec((NC, NROW, 8), lambda i: (0, i, 0)),
                  _row_spec(4), _row_spec(2), _row_spec(8),
                  _full_spec((16, 256)), _full_spec((1, 256)),
                  _full_spec((256, 256)), _full_spec((1, 256)),
                  _full_spec((256, 16)), _full_spec((1, 16)),
                  _full_spec((16, 8)), _full_spec((1, 8))],
        out_specs=[_row_spec(1)],
        out_shape=[jax.ShapeDtypeStruct((N, 1), f32)],
    )(p2, res2, hatt, inputs, w1p, b1p, w2p, b2p, w3p, b3p, w4p, b4p)

    return out[0]


# speedup vs baseline: 589.6283x; 1.0197x over previous
import functools

import jax
import jax.numpy as jnp
from jax import lax
from jax.experimental import pallas as pl
from jax.experimental.pallas import tpu as pltpu
from jax.experimental.pallas import tpu_sc as plsc

N = 100000
E = 3200000
NC, NS, L = 2, 16, 16
NW = NC * NS
SUB = 128
EPT = 100352
E_PAD = EPT * NW
N_ACC = 100352
RPT = N_ACC // NS
NROW = 3584
NGRID = 28


@functools.cache
def _mesh():
    return plsc.VectorSubcoreMesh(core_axis_name="c", subcore_axis_name="s",
                                  num_cores=NC, num_subcores=NS)


def _leaky02(x):
    return jnp.where(x > 0, x, 0.2 * x)


def _compute_group1(gs, gd, dpv, contrib, abv, g):
    rowv = lax.iota(jnp.int32, L) + g * L
    cols = [jnp.full((L,), f, jnp.int32) for f in range(16)]
    g1 = [plsc.load_gather(gs, [dpv, rowv, cols[f]]) for f in range(12)]
    g2 = [plsc.load_gather(gd, [dpv, rowv, cols[f]]) for f in range(12)]
    e = [_leaky02(g1[f] + g2[f]) for f in range(12)]
    la0 = abv[0] * e[0]
    la1 = abv[1] * e[1]
    ld0 = abv[2] * e[2]
    for f in range(3, 7):
        ld0 = ld0 + abv[f] * e[f]
    ld1 = abv[7] * e[7]
    for f in range(8, 12):
        ld1 = ld1 + abv[f] * e[f]
    wa0 = jnp.exp(la0)
    wa1 = jnp.exp(la1)
    wd0 = jnp.exp(ld0)
    wd1 = jnp.exp(ld1)
    out = [wa0, wa1, wa0 * g1[0], wa1 * g1[1], wd0, wd1]
    out += [wd0 * g1[2 + f] for f in range(5)]
    out += [wd1 * g1[7 + f] for f in range(5)]
    for ci in range(16):
        plsc.store_scatter(contrib, [rowv, cols[ci]], out[ci])


def _compute_group2(gs, gd, dpv, contrib, abv, g):
    rowv = lax.iota(jnp.int32, L) + g * L
    cols = [jnp.full((L,), f, jnp.int32) for f in range(8)]
    g1 = [plsc.load_gather(gs, [dpv, rowv, cols[f]]) for f in range(4)]
    g2 = [plsc.load_gather(gd, [dpv, rowv, cols[4 + f]]) for f in range(4)]
    e = [_leaky02(g1[f] + g2[f]) for f in range(4)]
    l0 = abv[0] * e[0] + abv[1] * e[1]
    l1 = abv[2] * e[2] + abv[3] * e[3]
    w0 = jnp.exp(l0)
    w1 = jnp.exp(l1)
    zero = jnp.zeros((L,), jnp.float32)
    out = [w0, w1, w0 * g1[0], w0 * g1[1], w1 * g1[2], w1 * g1[3],
           zero, zero]
    for ci in range(8):
        plsc.store_scatter(contrib, [rowv, cols[ci]], out[ci])


def _edge_body(W, compute_group, BLK, NSUB, NBLK):
    def body(tsrc_hbm, tdst_hbm, idx_hbm, ab_hbm, z_hbm, out_hbm,
             ib, gs, gd, contrib, abv, acc, isem, gsem, ssem):
        c = lax.axis_index("c")
        s = lax.axis_index("s")
        wid = c * NS + s

        pltpu.sync_copy(z_hbm, acc.at[pl.ds(s * RPT, RPT)])
        pltpu.sync_copy(ab_hbm, abv)
        plsc.subcore_barrier()
        base_row = wid * (EPT // SUB)
        pltpu.sync_copy(idx_hbm.at[pl.ds(base_row, NSUB)], ib.at[0])

        def drain_idx():
            pltpu.make_async_copy(
                idx_hbm.at[pl.ds(0, NSUB)], ib.at[0], isem).wait()

        def drain_gathers():
            pltpu.make_async_copy(
                tsrc_hbm.at[pl.ds(0, BLK)], gs.at[0], gsem).wait()
            pltpu.make_async_copy(
                tsrc_hbm.at[pl.ds(0, BLK)], gd.at[0], gsem).wait()

        def drain_scatters():
            pltpu.make_async_copy(
                tsrc_hbm.at[pl.ds(0, BLK)], contrib, ssem).wait()

        def step(i, carry):
            dp = lax.rem(i, 2)
            sl_cur = lax.rem(i, 3)
            sl_nxt = lax.rem(i + 1, 3)
            sl_prv = lax.rem(i + 2, 3)

            @pl.when(i >= 2)
            def _():
                drain_scatters()

            @pl.when(i >= 1)
            def _():
                drain_gathers()

            @pl.when(i <= NBLK - 1)
            def _():
                @pl.when(i >= 1)
                def _():
                    drain_idx()

                rown = base_row + jnp.minimum(i + 1, NBLK - 1) * NSUB
                pltpu.async_copy(idx_hbm.at[pl.ds(rown, NSUB)],
                                 ib.at[sl_nxt], isem)
                for q in range(NSUB):
                    pltpu.async_copy(
                        tsrc_hbm.at[ib.at[sl_cur, q, 0]],
                        gs.at[dp].at[pl.ds(q * SUB, SUB)], gsem)
                    pltpu.async_copy(
                        tdst_hbm.at[ib.at[sl_cur, q, 1]],
                        gd.at[dp].at[pl.ds(q * SUB, SUB)], gsem)

            @pl.when(i >= 1)
            def _():
                dpv = jnp.full((L,), 1 - dp, jnp.int32)

                def grp(g, cc):
                    compute_group(gs, gd, dpv, contrib, abv, g)
                    return cc

                for q in range(NSUB):
                    lax.fori_loop(q * (SUB // L), (q + 1) * (SUB // L),
                                  grp, 0)
                    pltpu.async_copy(contrib.at[pl.ds(q * SUB, SUB)],
                                     acc.at[ib.at[sl_prv, q, 1]], ssem,
                                     add=True)

            return carry

        lax.fori_loop(0, NBLK + 1, step, 0)
        drain_scatters()
        drain_idx()
        plsc.subcore_barrier()
        pltpu.sync_copy(acc.at[pl.ds(s * RPT, RPT)],
                        out_hbm.at[c].at[pl.ds(s * RPT, RPT)])

    return body


@functools.cache
def _sc_pass_kernel(W, nab, pass_id, blk):
    compute_group = _compute_group1 if pass_id == 1 else _compute_group2
    BLK, NSUB, NBLK = blk, blk // SUB, EPT // blk
    return pl.kernel(
        _edge_body(W, compute_group, BLK, NSUB, NBLK),
        out_type=jax.ShapeDtypeStruct((NC, N_ACC, W), jnp.float32),
        mesh=_mesh(),
        scratch_types=[
            pltpu.VMEM((3, NSUB, 2, SUB), jnp.int32),
            pltpu.VMEM((2, BLK, W), jnp.float32),
            pltpu.VMEM((2, BLK, W), jnp.float32),
            pltpu.VMEM((BLK, W), jnp.float32),
            pltpu.VMEM((nab, 16), jnp.float32),
            pltpu.VMEM_SHARED((N_ACC, W), jnp.float32),
            pltpu.SemaphoreType.DMA,
            pltpu.SemaphoreType.DMA,
            pltpu.SemaphoreType.DMA,
        ],
        compiler_params=pltpu.CompilerParams(
            needs_layout_passes=False, use_tc_tiling_on_sc=False),
    )


def _sc_pass1(tsrc, tdst, idx_sd, ab, z):
    return _sc_pass_kernel(16, 12, 1, 256)(tsrc, tdst, idx_sd, ab, z)


def _sc_pass2(t2, idx_sd, ab, z):
    return _sc_pass_kernel(8, 4, 2, 1024)(t2, t2, idx_sd, ab, z)


def _t1_body(x_ref, ws_ref, bs_ref, wd_ref, bd_ref, wr_ref, br_ref,
             tsrc_ref, tdst_ref, res_ref):
    x5 = x_ref[:, :5]
    tsrc_ref[...] = jnp.dot(x5, ws_ref[...],
                            preferred_element_type=jnp.float32) + bs_ref[...]
    tdst_ref[...] = jnp.dot(x5, wd_ref[...],
                            preferred_element_type=jnp.float32) + bd_ref[...]
    res_ref[...] = jnp.dot(x5, wr_ref[...],
                           preferred_element_type=jnp.float32) + br_ref[...]


def _elu(x):
    return jnp.where(x > 0, x, jnp.exp(x) - 1.0)


def _t2_body(p_ref, res_ref, wc_ref, bc_ref, hatt_ref, t2_ref, res2_ref):
    pa = p_ref[0] + p_ref[1]
    den_a = pa[:, 0:2] + 1e-9
    rst_a = pa[:, 2:4]
    den_d = pa[:, 4:6] + 1e-9
    rst_d = pa[:, 6:16]
    h_att = _elu(rst_a / den_a + res_ref[:, 0:2])
    den_dx = jnp.concatenate(
        [jnp.broadcast_to(den_d[:, 0:1], (NROW, 5)),
         jnp.broadcast_to(den_d[:, 1:2], (NROW, 5))], axis=1)
    h_def = _elu(rst_d / den_dx + res_ref[:, 2:12])
    hw = jnp.dot(h_def, wc_ref[...],
                 preferred_element_type=jnp.float32) + bc_ref[...]
    hatt_ref[...] = h_att
    t2_ref[...] = hw[:, 0:8]
    res2_ref[...] = hw[:, 8:12]


def _t3_body(p_ref, res2_ref, hatt_ref, x_ref,
             w1_ref, b1_ref, w2_ref, b2_ref, w3_ref, b3_ref, w4_ref, b4_ref,
             out_ref):
    pa = p_ref[0] + p_ref[1]
    den = pa[:, 0:2] + 1e-9
    rst = pa[:, 2:6]
    den_x = jnp.concatenate(
        [jnp.broadcast_to(den[:, 0:1], (NROW, 2)),
         jnp.broadcast_to(den[:, 1:2], (NROW, 2))], axis=1)
    h_def2 = _elu(rst / den_x + res2_ref[...])
    z = jnp.concatenate(
        [hatt_ref[...], h_def2, x_ref[...],
         jnp.zeros((NROW, 2), jnp.float32)], axis=1)

    def lk(v):
        return jnp.where(v > 0, v, 0.01 * v)

    h = lk(jnp.dot(z, w1_ref[...],
                   preferred_element_type=jnp.float32) + b1_ref[...])
    h = lk(jnp.dot(h, w2_ref[...],
                   preferred_element_type=jnp.float32) + b2_ref[...])
    h = lk(jnp.dot(h, w3_ref[...],
                   preferred_element_type=jnp.float32) + b3_ref[...])
    o = jnp.dot(h, w4_ref[...],
                preferred_element_type=jnp.float32) + b4_ref[...]
    out_ref[...] = 1.0 / (1.0 + jnp.exp(-o[:, 0:1]))


def _row_spec(w):
    return pl.BlockSpec((NROW, w), lambda i: (i, 0))


def _full_spec(shape):
    return pl.BlockSpec(shape, lambda i: tuple(0 for _ in shape))


def kernel(inputs, edge_index, params):
    p = params
    f32 = jnp.float32

    src = edge_index[0].astype(jnp.int32)
    dst = edge_index[1].astype(jnp.int32)
    pad = E_PAD - E
    src_r = jnp.concatenate([src, jnp.zeros((pad,), jnp.int32)]
                            ).reshape(E_PAD // SUB, SUB)
    dst_r = jnp.concatenate([dst, jnp.full((pad,), N, jnp.int32)]
                            ).reshape(E_PAD // SUB, SUB)
    idx_sd = jnp.stack([src_r, dst_r], axis=1)

    ws1 = jnp.concatenate([p['Ws_a'], p['Ws_d1'],
                           jnp.zeros((5, 4), f32)], axis=1)
    bs1 = jnp.concatenate([p['bs_a'], p['bs_d1'],
                           jnp.zeros((4,), f32)]).reshape(1, 16)
    wd1 = jnp.concatenate([p['Wd_a'], p['Wd_d1'],
                           jnp.zeros((5, 4), f32)], axis=1)
    bd1 = jnp.concatenate([p['bd_a'], p['bd_d1'],
                           jnp.zeros((4,), f32)]).reshape(1, 16)
    wr1 = jnp.concatenate([p['Wr_a'], p['Wr_d1']], axis=1)
    br1 = jnp.concatenate([p['br_a'], p['br_d1']]).reshape(1, 12)

    ab1 = jnp.concatenate([p['attn_a'].reshape(-1),
                           p['attn_d1'].reshape(-1)])
    ab1 = jnp.broadcast_to(ab1[:, None], (12, 16)).astype(f32)
    ab2 = jnp.broadcast_to(p['attn_d2'].reshape(-1)[:, None], (4, 16))

    wc = jnp.concatenate([p['Ws_d2'], p['Wd_d2'], p['Wr_d2']], axis=1)
    bc = jnp.concatenate([p['bs_d2'], p['bd_d2'], p['br_d2']]).reshape(1, 12)

    w1p = jnp.zeros((16, 256), f32).at[:14, :196].set(p['W1'])
    b1p = jnp.zeros((1, 256), f32).at[0, :196].set(p['b1'])
    w2p = jnp.zeros((256, 256), f32).at[:196, :196].set(p['W2'])
    b2p = jnp.zeros((1, 256), f32).at[0, :196].set(p['b2'])
    w3p = jnp.zeros((256, 16), f32).at[:196, :14].set(p['W3'])
    b3p = jnp.zeros((1, 16), f32).at[0, :14].set(p['b3'])
    w4p = jnp.zeros((16, 8), f32).at[:14, 0:1].set(p['W4'])
    b4p = jnp.zeros((1, 8), f32).at[0, 0:1].set(p['b4'])

    z16 = jnp.zeros((RPT, 16), f32)
    z8 = jnp.zeros((RPT, 8), f32)

    tsrc, tdst, res1 = pl.pallas_call(
        _t1_body,
        grid=(NGRID,),
        in_specs=[_row_spec(8), _full_spec((5, 16)), _full_spec((1, 16)),
                  _full_spec((5, 16)), _full_spec((1, 16)),
                  _full_spec((5, 12)), _full_spec((1, 12))],
        out_specs=[_row_spec(16), _row_spec(16), _row_spec(12)],
        out_shape=[jax.ShapeDtypeStruct((N_ACC, 16), f32),
                   jax.ShapeDtypeStruct((N_ACC, 16), f32),
                   jax.ShapeDtypeStruct((N, 12), f32)],
    )(inputs, ws1, bs1, wd1, bd1, wr1, br1)

    p1 = _sc_pass1(tsrc, tdst, idx_sd, ab1, z16)

    hatt, t2, res2 = pl.pallas_call(
        _t2_body,
        grid=(NGRID,),
        in_specs=[pl.BlockSpec((NC, NROW, 16), lambda i: (0, i, 0)),
                  _row_spec(12), _full_spec((10, 12)), _full_spec((1, 12))],
        out_specs=[_row_spec(2), _row_spec(8), _row_spec(4)],
        out_shape=[jax.ShapeDtypeStruct((N, 2), f32),
                   jax.ShapeDtypeStruct((N_ACC, 8), f32),
                   jax.ShapeDtypeStruct((N, 4), f32)],
    )(p1, res1, wc, bc)

    p2 = _sc_pass2(t2, idx_sd, ab2, z8)

    out = pl.pallas_call(
        _t3_body,
        grid=(NGRID,),
        in_specs=[pl.BlockSpec((NC, NROW, 8), lambda i: (0, i, 0)),
                  _row_spec(4), _row_spec(2), _row_spec(8),
                  _full_spec((16, 256)), _full_spec((1, 256)),
                  _full_spec((256, 256)), _full_spec((1, 256)),
                  _full_spec((256, 16)), _full_spec((1, 16)),
                  _full_spec((16, 8)), _full_spec((1, 8))],
        out_specs=[_row_spec(1)],
        out_shape=[jax.ShapeDtypeStruct((N, 1), f32)],
    )(p2, res2, hatt, inputs, w1p, b1p, w2p, b2p, w3p, b3p, w4p, b4p)

    return out[0]
